# Initial kernel scaffold; baseline (speedup 1.0000x reference)
import jax
import jax.numpy as jnp
from jax.experimental import pallas as pl


def kernel(x, edge_index, node_label_index, node_label, W0, b0, W1, b1, W2, b2, W3, b3, Wh, bh):
    raise NotImplementedError("write your pallas kernel here")


# speedup vs baseline: 7.6046x; 7.6046x over previous
import functools

import jax
import jax.numpy as jnp
from jax import lax
from jax.experimental import pallas as pl
from jax.experimental.pallas import tpu as pltpu
from jax.experimental.pallas import tpu_sc as plsc

_NC = 2
_NS = 16
_NW = _NC * _NS

_C = 80


def _make_propagate(n, e, d):
    per_w = e // _NW
    nch = per_w // _C
    n_pad = ((n + _NS * 128 - 1) // (_NS * 128)) * (_NS * 128)
    rpt = n_pad // _NS
    wb = 32
    nwb = rpt // wb
    mesh = plsc.VectorSubcoreMesh(
        core_axis_name="c", subcore_axis_name="s",
        num_cores=_NC, num_subcores=_NS)

    def body(h_hbm, src_hbm, dst_hbm, out_hbm,
             src_v, dst_v, buf, zbuf, acc, sem):
        cid = lax.axis_index("c")
        sid = lax.axis_index("s")
        wid = sid * _NC + cid

        def zb(i, carry):
            for j in range(d // 16):
                zbuf[i, pl.ds(j * 16, 16)] = jnp.zeros((16,), jnp.float32)
            return carry
        lax.fori_loop(0, wb, zb, 0)

        def zacc(t, carry):
            pltpu.sync_copy(zbuf, acc.at[pl.ds(sid * rpt + t * wb, wb)])
            return carry
        lax.fori_loop(0, nwb, zacc, 0)
        plsc.subcore_barrier()

        def chunk(k, carry):
            pltpu.sync_copy(src_hbm.at[wid, k], src_v.at[0])
            pltpu.sync_copy(dst_hbm.at[wid, k], dst_v.at[0])
            pltpu.async_copy(h_hbm.at[src_v.at[0]], buf, sem).wait()
            pltpu.sync_copy(buf, acc.at[dst_v.at[0]], add=True)
            return carry
        lax.fori_loop(0, nch, chunk, 0)
        plsc.subcore_barrier()

        def wback(t, carry):
            row = sid * rpt + t * wb
            pltpu.sync_copy(acc.at[pl.ds(row, wb)], zbuf)
            pltpu.sync_copy(zbuf, out_hbm.at[cid, pl.ds(row, wb)])
            return carry
        lax.fori_loop(0, nwb, wback, 0)

    return pl.kernel(
        body,
        out_type=jax.ShapeDtypeStruct((_NC, n_pad, d), jnp.float32),
        mesh=mesh,
        scratch_types=[
            pltpu.VMEM((1, _C), jnp.int32),
            pltpu.VMEM((1, _C), jnp.int32),
            pltpu.VMEM((_C, d), jnp.float32),
            pltpu.VMEM((wb, d), jnp.float32),
            pltpu.VMEM_SHARED((n_pad, d), jnp.float32),
            pltpu.SemaphoreType.DMA,
        ],
    )


def _make_row_gather(n, d, b):
    per_w = b // _NW
    mesh = plsc.VectorSubcoreMesh(
        core_axis_name="c", subcore_axis_name="s",
        num_cores=_NC, num_subcores=_NS)

    def body(table_hbm, idx_hbm, out_hbm, idx_v, rows_v, sem):
        wid = lax.axis_index("s") * _NC + lax.axis_index("c")
        base = wid * per_w
        pltpu.sync_copy(idx_hbm.at[pl.ds(base, per_w)], idx_v)
        pltpu.async_copy(table_hbm.at[idx_v], rows_v, sem).wait()
        pltpu.sync_copy(rows_v, out_hbm.at[pl.ds(base, per_w)])

    return pl.kernel(
        body,
        out_type=jax.ShapeDtypeStruct((b, d), jnp.float32),
        mesh=mesh,
        scratch_types=[
            pltpu.VMEM((per_w,), jnp.int32),
            pltpu.VMEM((per_w, d), jnp.float32),
            pltpu.SemaphoreType.DMA,
        ],
    )


_R = 2000


def _tc_step(accp, h, s, nrm, x, w, beff, iflag):
    n, d = h.shape

    def body(ap, h_ref, s_ref, nrm_ref, x_ref, w_ref, b_ref, f_ref,
             comb_ref, sn_ref, nrmn_ref, hn_ref):
        is0 = f_ref[0, 0] > 0.0
        accsum = ap[0] + ap[1]
        nrm_new = jnp.where(is0, lax.rsqrt(accsum[:, 0:1] + 1.0),
                            nrm_ref[...])
        comb = (accsum + h_ref[...]) * nrm_new + b_ref[...]
        r = jnp.maximum(comb, 0.0)
        s_new = jnp.where(is0, 0.0, s_ref[...] + r)
        base = jnp.where(is0, x_ref[...], s_new)
        comb_ref[...] = comb
        sn_ref[...] = s_new
        nrmn_ref[...] = nrm_new
        hn_ref[...] = jnp.dot(base, w_ref[...],
                              preferred_element_type=jnp.float32) * nrm_new

    row = pl.BlockSpec((_R, d), lambda i: (i, 0))
    col = pl.BlockSpec((_R, 1), lambda i: (i, 0))
    return pl.pallas_call(
        body,
        grid=(n // _R,),
        in_specs=[
            pl.BlockSpec((2, _R, d), lambda i: (0, i, 0)),
            row, row, col, row,
            pl.BlockSpec((d, d), lambda i: (0, 0)),
            pl.BlockSpec((1, d), lambda i: (0, 0)),
            pl.BlockSpec((1, 1), lambda i: (0, 0)),
        ],
        out_specs=[row, row, col, row],
        out_shape=[
            jax.ShapeDtypeStruct((n, d), jnp.float32),
            jax.ShapeDtypeStruct((n, d), jnp.float32),
            jax.ShapeDtypeStruct((n, 1), jnp.float32),
            jax.ShapeDtypeStruct((n, d), jnp.float32),
        ],
    )(accp, h, s, nrm, x, w, beff, iflag)


def _head(rows, wh, bh):
    b, d = rows.shape
    dout = wh.shape[1]

    def body(r_ref, w_ref, b_ref, out_ref):
        xr = r_ref[...]
        m = jnp.max(xr, axis=1, keepdims=True)
        lse = jnp.log(jnp.sum(jnp.exp(xr - m), axis=1, keepdims=True)) + m
        out_ref[...] = jnp.dot(xr - lse, w_ref[...],
                               preferred_element_type=jnp.float32) + b_ref[...]

    return pl.pallas_call(
        body,
        out_shape=jax.ShapeDtypeStruct((b, dout), jnp.float32),
    )(rows, wh, bh.reshape(1, dout))


def kernel(x, edge_index, node_label_index, node_label,
           W0, b0, W1, b1, W2, b2, W3, b3, Wh, bh):
    n, d = x.shape
    e = edge_index.shape[1]
    nl = node_label_index.shape[0]
    per_w = e // _NW
    nch = per_w // _C

    src = edge_index[0].reshape(_NW, nch, _C)
    dst = edge_index[1].reshape(_NW, nch, _C)

    prop = _make_propagate(n, e, d)

    wstack = jnp.stack([W0, W1, W2, W3, W3])
    beff = jnp.stack([0.0 * b0, 1.0 * b0, 1.0 * b1, 2.0 * b2, 3.0 * b3])
    beff = beff.reshape(5, 1, d)
    iflag = jnp.array([1.0, 0.0, 0.0, 0.0, 0.0],
                      jnp.float32).reshape(5, 1, 1)

    def step(carry, xs):
        h, s, nrm = carry
        w, b_, f_ = xs
        accp = prop(h, src, dst)
        comb, s2, nrm2, h2 = _tc_step(accp, h, s, nrm, x, w, b_, f_)
        return (h2, s2, nrm2), comb

    init = (jnp.ones((n, d), jnp.float32),
            jnp.zeros((n, d), jnp.float32),
            jnp.ones((n, 1), jnp.float32))
    _, combs = lax.scan(step, init, (wstack, beff, iflag))
    out3 = combs[4]

    nl_pad = ((nl + _NW * 8 - 1) // (_NW * 8)) * (_NW * 8)
    idx = jnp.concatenate(
        [node_label_index,
         jnp.zeros((nl_pad - nl,), node_label_index.dtype)])
    rows = _make_row_gather(n, d, nl_pad)(out3, idx)
    pred = _head(rows, Wh, bh)[:nl]
    return pred, node_label
